# baseline (speedup 1.0000x reference)
import functools

import jax
import jax.numpy as jnp
from jax import lax
from jax.experimental import pallas as pl
from jax.experimental.pallas import tpu as pltpu
from jax.experimental.pallas import tpu_sc as plsc

B = 16384
D = 64
NC = 2
NS = 16
NW = NC * NS
BPW = B // NW
C = 128
NCHUNK = BPW // C


def _sc_gather(a1, r1, a2, r2, ent, rel, off):
  mesh = plsc.VectorSubcoreMesh(core_axis_name="c", subcore_axis_name="s")
  row_t = jax.ShapeDtypeStruct((B, D), jnp.float32)

  @functools.partial(
      pl.kernel,
      out_type=[row_t] * 6,
      mesh=mesh,
      compiler_params=pltpu.CompilerParams(use_tc_tiling_on_sc=False),
      scratch_types=[
          pltpu.VMEM((BPW,), jnp.int32),
          pltpu.VMEM((BPW,), jnp.int32),
          pltpu.VMEM((BPW,), jnp.int32),
          pltpu.VMEM((BPW,), jnp.int32),
          pltpu.VMEM((C, D), jnp.float32),
          pltpu.VMEM((C, D), jnp.float32),
          pltpu.VMEM((C, D), jnp.float32),
          pltpu.VMEM((C, D), jnp.float32),
          pltpu.VMEM((C, D), jnp.float32),
          pltpu.VMEM((C, D), jnp.float32),
          pltpu.SemaphoreType.DMA,
      ],
  )
  def k(a1_h, r1_h, a2_h, r2_h, ent_h, rel_h, off_h,
        e1_o, r1_o, e2_o, r2_o, o1_o, o2_o,
        ia1, ia2, ir1, ir2, be1, be2, br1, br2, bo1, bo2, sem):
    wid = lax.axis_index("s") * NC + lax.axis_index("c")
    base_w = wid * BPW
    pltpu.sync_copy(a1_h.at[pl.ds(base_w, BPW)], ia1)
    pltpu.sync_copy(a2_h.at[pl.ds(base_w, BPW)], ia2)
    pltpu.sync_copy(r1_h.at[pl.ds(base_w, BPW)], ir1)
    pltpu.sync_copy(r2_h.at[pl.ds(base_w, BPW)], ir2)
    for ci in range(NCHUNK):
      o = ci * C
      base = base_w + o
      cps = [
          pltpu.async_copy(ent_h.at[ia1.at[pl.ds(o, C)]], be1, sem),
          pltpu.async_copy(ent_h.at[ia2.at[pl.ds(o, C)]], be2, sem),
          pltpu.async_copy(rel_h.at[ir1.at[pl.ds(o, C)]], br1, sem),
          pltpu.async_copy(rel_h.at[ir2.at[pl.ds(o, C)]], br2, sem),
          pltpu.async_copy(off_h.at[ir1.at[pl.ds(o, C)]], bo1, sem),
          pltpu.async_copy(off_h.at[ir2.at[pl.ds(o, C)]], bo2, sem),
      ]
      for cp in cps:
        cp.wait()
      pltpu.sync_copy(be1, e1_o.at[pl.ds(base, C)])
      pltpu.sync_copy(be2, e2_o.at[pl.ds(base, C)])
      pltpu.sync_copy(br1, r1_o.at[pl.ds(base, C)])
      pltpu.sync_copy(br2, r2_o.at[pl.ds(base, C)])
      pltpu.sync_copy(bo1, o1_o.at[pl.ds(base, C)])
      pltpu.sync_copy(bo2, o2_o.at[pl.ds(base, C)])

  return k(a1, r1, a2, r2, ent, rel, off)


BK = 2048


def _tc_body(e1, r1, e2, r2, o1, o2,
             cw1, cb1, cw2, cb2, ow1, ob1, ow2, ob2,
             center_o, offset_o):
  dn = (((1,), (1,)), ((), ()))
  emb1 = e1[...] + r1[...]
  emb2 = e2[...] + r2[...]
  l11 = jnp.maximum(
      lax.dot_general(emb1, cw1[...], dn, preferred_element_type=jnp.float32)
      + cb1[...], 0.0)
  l12 = jnp.maximum(
      lax.dot_general(emb2, cw1[...], dn, preferred_element_type=jnp.float32)
      + cb1[...], 0.0)
  a1 = lax.dot_general(l11, cw2[...], dn,
                       preferred_element_type=jnp.float32) + cb2[...]
  a2 = lax.dot_general(l12, cw2[...], dn,
                       preferred_element_type=jnp.float32) + cb2[...]
  m = jnp.maximum(a1, a2)
  x1 = jnp.exp(a1 - m)
  x2 = jnp.exp(a2 - m)
  center_o[...] = (x1 * emb1 + x2 * emb2) / (x1 + x2)

  ob1v = ob1[...]
  oa1 = jnp.maximum(
      lax.dot_general(o1[...], ow1[...], dn,
                      preferred_element_type=jnp.float32) + ob1v, 0.0)
  oa2 = jnp.maximum(
      lax.dot_general(o2[...], ow1[...], dn,
                      preferred_element_type=jnp.float32) + ob1v, 0.0)
  omean = 0.5 * (oa1 + oa2)
  gate = jax.nn.sigmoid(
      lax.dot_general(omean, ow2[...], dn,
                      preferred_element_type=jnp.float32) + ob2[...])
  offset_o[...] = jnp.minimum(o1[...], o2[...]) * gate


def _tc_intersect(e1, r1, e2, r2, o1, o2,
                  cw1, cb1, cw2, cb2, ow1, ob1, ow2, ob2):
  rows = pl.BlockSpec((BK, D), lambda i: (i, 0))
  wmat = pl.BlockSpec((D, D), lambda i: (0, 0))
  wvec = pl.BlockSpec((1, D), lambda i: (0, 0))
  return pl.pallas_call(
      _tc_body,
      grid=(B // BK,),
      in_specs=[rows] * 6 + [wmat, wvec, wmat, wvec, wmat, wvec, wmat, wvec],
      out_specs=[rows, rows],
      out_shape=[jax.ShapeDtypeStruct((B, D), jnp.float32)] * 2,
  )(e1, r1, e2, r2, o1, o2, cw1, cb1, cw2, cb2, ow1, ob1, ow2, ob2)


def kernel(anchor1, rel1, anchor2, rel2, entity_embedding, relation_embedding,
           offset_embedding, c_w1, c_b1, c_w2, c_b2, o_w1, o_b1, o_w2, o_b2):
  a1 = anchor1.astype(jnp.int32)
  a2 = anchor2.astype(jnp.int32)
  r1 = rel1.astype(jnp.int32)
  r2 = rel2.astype(jnp.int32)
  e1, rr1, e2, rr2, o1, o2 = _sc_gather(
      a1, r1, a2, r2, entity_embedding, relation_embedding, offset_embedding)
  center, offset = _tc_intersect(
      e1, rr1, e2, rr2, o1, o2,
      c_w1, c_b1.reshape(1, D), c_w2, c_b2.reshape(1, D),
      o_w1, o_b1.reshape(1, D), o_w2, o_b2.reshape(1, D))
  return (center, offset)


# speedup vs baseline: 1.0132x; 1.0132x over previous
import functools

import jax
import jax.numpy as jnp
from jax import lax
from jax.experimental import pallas as pl
from jax.experimental.pallas import tpu as pltpu
from jax.experimental.pallas import tpu_sc as plsc

B = 16384
D = 64
NC = 2
NS = 16
NW = NC * NS
BPW = B // NW
C = 128
NCHUNK = BPW // C
NV = D // 16


def _sc_gather(a1, r1, a2, r2, ent, rel, off):
  mesh = plsc.VectorSubcoreMesh(core_axis_name="c", subcore_axis_name="s")
  out_t = jax.ShapeDtypeStruct((B, 2 * D), jnp.float32)

  @functools.partial(
      pl.kernel,
      out_type=[out_t, out_t],
      mesh=mesh,
      compiler_params=pltpu.CompilerParams(use_tc_tiling_on_sc=False),
      scratch_types=[
          pltpu.VMEM((BPW,), jnp.int32),
          pltpu.VMEM((BPW,), jnp.int32),
          pltpu.VMEM((BPW,), jnp.int32),
          pltpu.VMEM((BPW,), jnp.int32),
          pltpu.VMEM((C, D), jnp.float32),
          pltpu.VMEM((C, D), jnp.float32),
          pltpu.VMEM((C, D), jnp.float32),
          pltpu.VMEM((C, D), jnp.float32),
          pltpu.VMEM((C, D), jnp.float32),
          pltpu.VMEM((C, D), jnp.float32),
          pltpu.VMEM((C, 2 * D), jnp.float32),
          pltpu.VMEM((C, 2 * D), jnp.float32),
          pltpu.SemaphoreType.DMA,
      ],
  )
  def k(a1_h, r1_h, a2_h, r2_h, ent_h, rel_h, off_h,
        emb_o, off_o,
        ia1, ia2, ir1, ir2, be1, be2, br1, br2, bo1, bo2, pe, po, sem):
    wid = lax.axis_index("s") * NC + lax.axis_index("c")
    base_w = wid * BPW
    pltpu.sync_copy(a1_h.at[pl.ds(base_w, BPW)], ia1)
    pltpu.sync_copy(a2_h.at[pl.ds(base_w, BPW)], ia2)
    pltpu.sync_copy(r1_h.at[pl.ds(base_w, BPW)], ir1)
    pltpu.sync_copy(r2_h.at[pl.ds(base_w, BPW)], ir2)
    for ci in range(NCHUNK):
      o = ci * C
      base = base_w + o
      cps = [
          pltpu.async_copy(ent_h.at[ia1.at[pl.ds(o, C)]], be1, sem),
          pltpu.async_copy(ent_h.at[ia2.at[pl.ds(o, C)]], be2, sem),
          pltpu.async_copy(rel_h.at[ir1.at[pl.ds(o, C)]], br1, sem),
          pltpu.async_copy(rel_h.at[ir2.at[pl.ds(o, C)]], br2, sem),
          pltpu.async_copy(off_h.at[ir1.at[pl.ds(o, C)]], bo1, sem),
          pltpu.async_copy(off_h.at[ir2.at[pl.ds(o, C)]], bo2, sem),
      ]
      for cp in cps:
        cp.wait()

      def row_body(rr, _):
        for j in range(NV):
          sl = pl.ds(j * 16, 16)
          sr = pl.ds(D + j * 16, 16)
          pe[rr, sl] = be1[rr, sl] + br1[rr, sl]
          pe[rr, sr] = be2[rr, sl] + br2[rr, sl]
          po[rr, sl] = bo1[rr, sl]
          po[rr, sr] = bo2[rr, sl]
        return 0

      lax.fori_loop(0, C, row_body, 0, unroll=2)
      pltpu.sync_copy(pe, emb_o.at[pl.ds(base, C)])
      pltpu.sync_copy(po, off_o.at[pl.ds(base, C)])

  return k(a1, r1, a2, r2, ent, rel, off)


BK = 2048


def _tc_body(embs, offs, cw1, cb1, cw2, cb2, ow1, ob1, ow2, ob2,
             center_o, offset_o):
  dn = (((1,), (1,)), ((), ()))
  emb1 = embs[:, :D]
  emb2 = embs[:, D:]
  cb1v = cb1[...]
  cb2v = cb2[...]
  l11 = jnp.maximum(
      lax.dot_general(emb1, cw1[...], dn, preferred_element_type=jnp.float32)
      + cb1v, 0.0)
  l12 = jnp.maximum(
      lax.dot_general(emb2, cw1[...], dn, preferred_element_type=jnp.float32)
      + cb1v, 0.0)
  a1 = lax.dot_general(l11, cw2[...], dn,
                       preferred_element_type=jnp.float32) + cb2v
  a2 = lax.dot_general(l12, cw2[...], dn,
                       preferred_element_type=jnp.float32) + cb2v
  m = jnp.maximum(a1, a2)
  x1 = jnp.exp(a1 - m)
  x2 = jnp.exp(a2 - m)
  center_o[...] = (x1 * emb1 + x2 * emb2) / (x1 + x2)

  o1 = offs[:, :D]
  o2 = offs[:, D:]
  ob1v = ob1[...]
  oa1 = jnp.maximum(
      lax.dot_general(o1, ow1[...], dn,
                      preferred_element_type=jnp.float32) + ob1v, 0.0)
  oa2 = jnp.maximum(
      lax.dot_general(o2, ow1[...], dn,
                      preferred_element_type=jnp.float32) + ob1v, 0.0)
  omean = 0.5 * (oa1 + oa2)
  gate = jax.nn.sigmoid(
      lax.dot_general(omean, ow2[...], dn,
                      preferred_element_type=jnp.float32) + ob2[...])
  offset_o[...] = jnp.minimum(o1, o2) * gate


def _tc_intersect(embs, offs, cw1, cb1, cw2, cb2, ow1, ob1, ow2, ob2):
  rows2 = pl.BlockSpec((BK, 2 * D), lambda i: (i, 0))
  rows = pl.BlockSpec((BK, D), lambda i: (i, 0))
  wmat = pl.BlockSpec((D, D), lambda i: (0, 0))
  wvec = pl.BlockSpec((1, D), lambda i: (0, 0))
  return pl.pallas_call(
      _tc_body,
      grid=(B // BK,),
      in_specs=[rows2, rows2, wmat, wvec, wmat, wvec, wmat, wvec, wmat, wvec],
      out_specs=[rows, rows],
      out_shape=[jax.ShapeDtypeStruct((B, D), jnp.float32)] * 2,
  )(embs, offs, cw1, cb1, cw2, cb2, ow1, ob1, ow2, ob2)


def kernel(anchor1, rel1, anchor2, rel2, entity_embedding, relation_embedding,
           offset_embedding, c_w1, c_b1, c_w2, c_b2, o_w1, o_b1, o_w2, o_b2):
  a1 = anchor1.astype(jnp.int32)
  a2 = anchor2.astype(jnp.int32)
  r1 = rel1.astype(jnp.int32)
  r2 = rel2.astype(jnp.int32)
  embs, offs = _sc_gather(
      a1, r1, a2, r2, entity_embedding, relation_embedding, offset_embedding)
  center, offset = _tc_intersect(
      embs, offs,
      c_w1, c_b1.reshape(1, D), c_w2, c_b2.reshape(1, D),
      o_w1, o_b1.reshape(1, D), o_w2, o_b2.reshape(1, D))
  return (center, offset)


# speedup vs baseline: 1.5327x; 1.5127x over previous
import functools

import jax
import jax.numpy as jnp
from jax import lax
from jax.experimental import pallas as pl
from jax.experimental.pallas import tpu as pltpu
from jax.experimental.pallas import tpu_sc as plsc

B = 16384
D = 64
NC = 2
NS = 16
NW = NC * NS
BPW = B // NW
C = 64
NCHUNK = BPW // C
NV = D // 16


def _sc_gather(a1, r1, a2, r2, ent, relp, offp):
  mesh = plsc.VectorSubcoreMesh(core_axis_name="c", subcore_axis_name="s")
  out_t = jax.ShapeDtypeStruct((B, 2 * D), jnp.float32)

  @functools.partial(
      pl.kernel,
      out_type=[out_t, out_t],
      mesh=mesh,
      compiler_params=pltpu.CompilerParams(needs_layout_passes=False),
      scratch_types=[
          pltpu.VMEM((BPW,), jnp.int32),
          pltpu.VMEM((BPW,), jnp.int32),
          pltpu.VMEM((BPW,), jnp.int32),
          pltpu.VMEM((BPW,), jnp.int32),
          pltpu.VMEM((C, D), jnp.float32),
          pltpu.VMEM((C, D), jnp.float32),
          pltpu.VMEM((C, 2 * D), jnp.float32),
          pltpu.VMEM((C, 2 * D), jnp.float32),
          pltpu.VMEM((C, 2 * D), jnp.float32),
          pltpu.VMEM((C, 2 * D), jnp.float32),
          pltpu.VMEM((C, 2 * D), jnp.float32),
          pltpu.VMEM((C, 2 * D), jnp.float32),
          pltpu.SemaphoreType.DMA,
      ],
  )
  def k(a1_h, r1_h, a2_h, r2_h, ent_h, rel_h, off_h,
        emb_o, off_o,
        ia1, ia2, ir1, ir2, eb1, eb2, rb1, rb2, ob1, ob2, pe, po, sem):
    wid = lax.axis_index("s") * NC + lax.axis_index("c")
    base_w = wid * BPW
    pltpu.sync_copy(a1_h.at[pl.ds(base_w, BPW)], ia1)
    pltpu.sync_copy(a2_h.at[pl.ds(base_w, BPW)], ia2)
    pltpu.sync_copy(r1_h.at[pl.ds(base_w, BPW)], ir1)
    pltpu.sync_copy(r2_h.at[pl.ds(base_w, BPW)], ir2)
    for ci in range(NCHUNK):
      o = ci * C
      gcps = [
          pltpu.async_copy(rel_h.at[ir1.at[pl.ds(o, C)]], rb1, sem),
          pltpu.async_copy(rel_h.at[ir2.at[pl.ds(o, C)]], rb2, sem),
          pltpu.async_copy(off_h.at[ir1.at[pl.ds(o, C)]], ob1, sem),
          pltpu.async_copy(off_h.at[ir2.at[pl.ds(o, C)]], ob2, sem),
      ]

      def issue(g, _):
        v1 = ia1[pl.ds(o + g * 16, 16)]
        v2 = ia2[pl.ds(o + g * 16, 16)]
        for l in range(16):
          q = g * 16 + l
          pltpu.async_copy(ent_h.at[pl.ds(v1[l], 1)],
                           eb1.at[pl.ds(q, 1)], sem)
          pltpu.async_copy(ent_h.at[pl.ds(v2[l], 1)],
                           eb2.at[pl.ds(q, 1)], sem)
        return 0

      lax.fori_loop(0, C // 16, issue, 0)
      for cp in gcps:
        cp.wait()
      for _ in range(2 * C):
        pltpu.make_async_copy(ent_h.at[pl.ds(0, 1)], eb1.at[pl.ds(0, 1)],
                              sem).wait()

      def row_body(rr, _):
        for j in range(NV):
          sl = pl.ds(j * 16, 16)
          sr = pl.ds(D + j * 16, 16)
          pe[rr, sl] = eb1[rr, sl] + rb1[rr, sl]
          pe[rr, sr] = eb2[rr, sl] + rb2[rr, sl]
          po[rr, sl] = ob1[rr, sl]
          po[rr, sr] = ob2[rr, sl]
        return 0

      lax.fori_loop(0, C, row_body, 0, unroll=2)
      base = base_w + o
      pltpu.sync_copy(pe, emb_o.at[pl.ds(base, C)])
      pltpu.sync_copy(po, off_o.at[pl.ds(base, C)])

  return k(a1, r1, a2, r2, ent, relp, offp)


BK = 2048


def _tc_body(embs, offs, cw1, cb1, cw2, cb2, ow1, ob1, ow2, ob2,
             center_o, offset_o):
  dn = (((1,), (1,)), ((), ()))
  emb1 = embs[:, :D]
  emb2 = embs[:, D:]
  cb1v = cb1[...]
  cb2v = cb2[...]
  l11 = jnp.maximum(
      lax.dot_general(emb1, cw1[...], dn, preferred_element_type=jnp.float32)
      + cb1v, 0.0)
  l12 = jnp.maximum(
      lax.dot_general(emb2, cw1[...], dn, preferred_element_type=jnp.float32)
      + cb1v, 0.0)
  a1 = lax.dot_general(l11, cw2[...], dn,
                       preferred_element_type=jnp.float32) + cb2v
  a2 = lax.dot_general(l12, cw2[...], dn,
                       preferred_element_type=jnp.float32) + cb2v
  m = jnp.maximum(a1, a2)
  x1 = jnp.exp(a1 - m)
  x2 = jnp.exp(a2 - m)
  center_o[...] = (x1 * emb1 + x2 * emb2) / (x1 + x2)

  o1 = offs[:, :D]
  o2 = offs[:, D:]
  ob1v = ob1[...]
  oa1 = jnp.maximum(
      lax.dot_general(o1, ow1[...], dn,
                      preferred_element_type=jnp.float32) + ob1v, 0.0)
  oa2 = jnp.maximum(
      lax.dot_general(o2, ow1[...], dn,
                      preferred_element_type=jnp.float32) + ob1v, 0.0)
  omean = 0.5 * (oa1 + oa2)
  gate = jax.nn.sigmoid(
      lax.dot_general(omean, ow2[...], dn,
                      preferred_element_type=jnp.float32) + ob2[...])
  offset_o[...] = jnp.minimum(o1, o2) * gate


def _tc_intersect(embs, offs, cw1, cb1, cw2, cb2, ow1, ob1, ow2, ob2):
  rows2 = pl.BlockSpec((BK, 2 * D), lambda i: (i, 0))
  rows = pl.BlockSpec((BK, D), lambda i: (i, 0))
  wmat = pl.BlockSpec((D, D), lambda i: (0, 0))
  wvec = pl.BlockSpec((1, D), lambda i: (0, 0))
  return pl.pallas_call(
      _tc_body,
      grid=(B // BK,),
      in_specs=[rows2, rows2, wmat, wvec, wmat, wvec, wmat, wvec, wmat, wvec],
      out_specs=[rows, rows],
      out_shape=[jax.ShapeDtypeStruct((B, D), jnp.float32)] * 2,
  )(embs, offs, cw1, cb1, cw2, cb2, ow1, ob1, ow2, ob2)


def kernel(anchor1, rel1, anchor2, rel2, entity_embedding, relation_embedding,
           offset_embedding, c_w1, c_b1, c_w2, c_b2, o_w1, o_b1, o_w2, o_b2):
  a1 = anchor1.astype(jnp.int32)
  a2 = anchor2.astype(jnp.int32)
  r1 = rel1.astype(jnp.int32)
  r2 = rel2.astype(jnp.int32)
  relp = jnp.pad(relation_embedding, ((0, 0), (0, D)))
  offp = jnp.pad(offset_embedding, ((0, 0), (0, D)))
  embs, offs = _sc_gather(a1, r1, a2, r2, entity_embedding, relp, offp)
  center, offset = _tc_intersect(
      embs, offs,
      c_w1, c_b1.reshape(1, D), c_w2, c_b2.reshape(1, D),
      o_w1, o_b1.reshape(1, D), o_w2, o_b2.reshape(1, D))
  return (center, offset)
